# baseline (speedup 1.0000x reference)
import functools

import jax
import jax.numpy as jnp
from jax import lax
from jax.experimental import pallas as pl
from jax.experimental.pallas import tpu as pltpu
from jax.experimental.pallas import tpu_sc as plsc

_BATCH = 4
_SEQ = 8192
_D = 768
_NW = 32
_ROWS_PER_W = _SEQ // _NW
_R = 8
_N_CHUNKS = _ROWS_PER_W // _R
_L = 16
_NSLOT = 4


def _sc_body(x_hbm, t_hbm, out_hbm, *scratch):
    bufx = scratch[:_NSLOT]
    buft = scratch[_NSLOT:_NSLOT + 2]
    sx = scratch[_NSLOT + 2:2 * _NSLOT + 2]
    st = scratch[2 * _NSLOT + 2:2 * _NSLOT + 4]
    so = scratch[2 * _NSLOT + 4:3 * _NSLOT + 4]

    wid = lax.axis_index("s") * 2 + lax.axis_index("c")
    r0 = wid * _ROWS_PER_W

    def x_copy(g, s):
        row = r0 + g * _R
        return pltpu.make_async_copy(
            x_hbm.at[:, pl.ds(row, _R), :], bufx[s], sx[s])

    def t_copy(g, s):
        row = r0 + g * _R
        return pltpu.make_async_copy(
            t_hbm.at[pl.ds(row, _R), :], buft[s], st[s])

    def out_copy(g, s):
        row = r0 + g * _R
        return pltpu.make_async_copy(
            bufx[s], out_hbm.at[:, pl.ds(row, _R), :], so[s])

    def accumulate(s, ts):
        half = _D // 2

        @plsc.parallel_loop(0, 2 * _R * _BATCH, 1, unroll=2)
        def _(idx):
            bb = idx >> 4
            r2 = idx & (2 * _R - 1)
            r = r2 >> 1
            c0 = (r2 & 1) * half
            vs = [buft[ts][r, pl.ds(c0 + j * _L, _L)]
                  for j in range(half // _L)]
            for j, v in enumerate(vs):
                plsc.addupdate(bufx[s].at[bb, r, pl.ds(c0 + j * _L, _L)], v)

    t_copy(0, 0).start()
    x_copy(0, 0).start()
    x_copy(1, 1).start()

    def block(i, carry):
        for gp in range(4):
            g = 4 * i + gp
            s = gp
            ps = (gp + 2) % 4

            @pl.when(g + 1 < _N_CHUNKS)
            def _():
                t_copy(g + 1, (gp + 1) % 2).start()

            t_copy(g, gp % 2).wait()

            @pl.when(g >= 2)
            def _():
                out_copy(g - 2, ps).wait()

            @pl.when(g + 2 < _N_CHUNKS)
            def _():
                x_copy(g + 2, ps).start()

            x_copy(g, s).wait()
            accumulate(s, gp % 2)
            out_copy(g, s).start()
        return carry

    lax.fori_loop(0, _N_CHUNKS // 4, block, 0)
    out_copy(_N_CHUNKS - 2, 2).wait()
    out_copy(_N_CHUNKS - 1, 3).wait()


_sc_add = functools.partial(
    pl.kernel,
    out_type=jax.ShapeDtypeStruct((_BATCH, _SEQ, _D), jnp.float32),
    mesh=plsc.VectorSubcoreMesh(core_axis_name="c", subcore_axis_name="s"),
    compiler_params=pltpu.CompilerParams(use_tc_tiling_on_sc=True),
    scratch_types=(
        [pltpu.VMEM((_BATCH, _R, _D), jnp.float32)] * _NSLOT
        + [pltpu.VMEM((_R, _D), jnp.float32)] * 2
        + [pltpu.SemaphoreType.DMA] * (2 * _NSLOT + 2)
    ),
)(_sc_body)


def kernel(x, pos_table):
    return _sc_add(x, pos_table)
